# Initial kernel scaffold; baseline (speedup 1.0000x reference)
import jax
import jax.numpy as jnp
from jax.experimental import pallas as pl


def kernel(atom_features, edge_indices, W_emb, b_emb, msg_W, msg_b, upd_W, upd_b, Wo1, bo1, Wo2, bo2, Wo3, bo3):
    raise NotImplementedError("write your pallas kernel here")


# speedup vs baseline: 6.8243x; 6.8243x over previous
import functools

import jax
import jax.numpy as jnp
from jax import lax
from jax.experimental import pallas as pl
from jax.experimental.pallas import tpu as pltpu
from jax.experimental.pallas import tpu_sc as plsc

N = 10000
E = 320000
D = 128
NC = 2
NS = 16
NW = NC * NS
EPW = E // NW
CH = 128
NFULL = EPW // CH
TAIL = EPW - NFULL * CH
RPT = 624
REM = N - NS * RPT

@functools.cache
def _mesh():
    return plsc.VectorSubcoreMesh(
        core_axis_name="c", subcore_axis_name="s",
        num_cores=NC, num_subcores=NS)


def _seg_body(h_hbm, src_hbm, dst_hbm, z_hbm, out_hbm,
              sidx, didx, sidx_t, didx_t, rows, rows_t, acc, sem):
    c = lax.axis_index("c")
    s = lax.axis_index("s")
    wid = s * NC + c
    pltpu.sync_copy(z_hbm.at[pl.ds(0, RPT)], acc.at[pl.ds(s * RPT, RPT)])

    @pl.when(s == 0)
    def _():
        pltpu.sync_copy(z_hbm.at[pl.ds(0, REM)], acc.at[pl.ds(NS * RPT, REM)])
    plsc.subcore_barrier()
    base = wid * EPW

    def chunk(j, carry):
        off = base + j * CH
        pltpu.sync_copy(src_hbm.at[pl.ds(off, CH)], sidx)
        pltpu.sync_copy(dst_hbm.at[pl.ds(off, CH)], didx)
        pltpu.async_copy(h_hbm.at[sidx], rows, sem).wait()
        pltpu.sync_copy(rows, acc.at[didx], add=True)
        return carry

    lax.fori_loop(0, NFULL, chunk, 0)
    off_t = base + NFULL * CH
    pltpu.sync_copy(src_hbm.at[pl.ds(off_t, TAIL)], sidx_t)
    pltpu.sync_copy(dst_hbm.at[pl.ds(off_t, TAIL)], didx_t)
    pltpu.async_copy(h_hbm.at[sidx_t], rows_t, sem).wait()
    pltpu.sync_copy(rows_t, acc.at[didx_t], add=True)
    plsc.subcore_barrier()
    pltpu.sync_copy(acc.at[pl.ds(s * RPT, RPT)],
                    out_hbm.at[pl.ds(c * N + s * RPT, RPT)])

    @pl.when(s == 0)
    def _():
        pltpu.sync_copy(acc.at[pl.ds(NS * RPT, REM)],
                        out_hbm.at[pl.ds(c * N + NS * RPT, REM)])


def _segment_partials(h, src, dst, zeros_rows):
    return pl.kernel(
        _seg_body,
        out_type=jax.ShapeDtypeStruct((2 * N, D), jnp.float32),
        mesh=_mesh(),
        scratch_types=[
            pltpu.VMEM((CH,), jnp.int32),
            pltpu.VMEM((CH,), jnp.int32),
            pltpu.VMEM((TAIL,), jnp.int32),
            pltpu.VMEM((TAIL,), jnp.int32),
            pltpu.VMEM((CH, D), jnp.float32),
            pltpu.VMEM((TAIL, D), jnp.float32),
            pltpu.VMEM_SHARED((N, D), jnp.float32),
            pltpu.SemaphoreType.DMA,
        ],
    )(h, src, dst, zeros_rows)


def _deg_body(dst_hbm, z_hbm, ones_hbm, out_hbm, didx, didx_t, ones, acc, sem):
    c = lax.axis_index("c")
    s = lax.axis_index("s")
    wid = s * NC + c
    pltpu.sync_copy(ones_hbm, ones)
    pltpu.sync_copy(z_hbm.at[pl.ds(0, RPT)], acc.at[pl.ds(s * RPT, RPT)])

    @pl.when(s == 0)
    def _():
        pltpu.sync_copy(z_hbm.at[pl.ds(0, REM)], acc.at[pl.ds(NS * RPT, REM)])
    plsc.subcore_barrier()
    base = wid * EPW

    def chunk(j, carry):
        off = base + j * CH
        pltpu.sync_copy(dst_hbm.at[pl.ds(off, CH)], didx)
        pltpu.sync_copy(ones, acc.at[didx], add=True)
        return carry

    lax.fori_loop(0, NFULL, chunk, 0)
    off_t = base + NFULL * CH
    pltpu.sync_copy(dst_hbm.at[pl.ds(off_t, TAIL)], didx_t)
    pltpu.sync_copy(ones.at[pl.ds(0, TAIL)], acc.at[didx_t], add=True)
    plsc.subcore_barrier()
    pltpu.sync_copy(acc.at[pl.ds(s * RPT, RPT)],
                    out_hbm.at[pl.ds(c * N + s * RPT, RPT)])

    @pl.when(s == 0)
    def _():
        pltpu.sync_copy(acc.at[pl.ds(NS * RPT, REM)],
                        out_hbm.at[pl.ds(c * N + NS * RPT, REM)])


def _deg_partials(dst, zeros_rows, ones_rows):
    return pl.kernel(
        _deg_body,
        out_type=jax.ShapeDtypeStruct((2 * N, D), jnp.float32),
        mesh=_mesh(),
        scratch_types=[
            pltpu.VMEM((CH,), jnp.int32),
            pltpu.VMEM((TAIL,), jnp.int32),
            pltpu.VMEM((CH, D), jnp.float32),
            pltpu.VMEM_SHARED((N, D), jnp.float32),
            pltpu.SemaphoreType.DMA,
        ],
    )(dst, zeros_rows, ones_rows)


BLK = 1000
GRID = N // BLK


def _emb_body(x_ref, w_ref, b_ref, o_ref):
    o_ref[...] = (jnp.dot(x_ref[...], w_ref[...],
                          preferred_element_type=jnp.float32,
                          precision=lax.Precision.HIGHEST) + b_ref[...])


def _embed(x, W, b2d):
    return pl.pallas_call(
        _emb_body,
        grid=(GRID,),
        in_specs=[
            pl.BlockSpec((BLK, D), lambda i: (i, 0)),
            pl.BlockSpec((D, D), lambda i: (0, 0)),
            pl.BlockSpec((1, D), lambda i: (0, 0)),
        ],
        out_specs=pl.BlockSpec((BLK, D), lambda i: (i, 0)),
        out_shape=jax.ShapeDtypeStruct((N, D), jnp.float32),
    )(x, W, b2d)


def _layer_body(h_ref, s0_ref, s1_ref, d0_ref, d1_ref,
                ws_ref, wd_ref, u1_ref, u2_ref, mb_ref, ub_ref, o_ref):
    h = h_ref[...]
    S = s0_ref[...] + s1_ref[...]
    d = d0_ref[:, 0:1] + d1_ref[:, 0:1]
    agg = (jnp.dot(S, ws_ref[...], preferred_element_type=jnp.float32,
                          precision=lax.Precision.HIGHEST)
           + jnp.dot(h * d, wd_ref[...], preferred_element_type=jnp.float32,
                          precision=lax.Precision.HIGHEST)
           + d * mb_ref[...])
    hn = (jnp.dot(h, u1_ref[...], preferred_element_type=jnp.float32,
                          precision=lax.Precision.HIGHEST)
          + jnp.dot(agg, u2_ref[...], preferred_element_type=jnp.float32,
                          precision=lax.Precision.HIGHEST)
          + ub_ref[...])
    o_ref[...] = jnp.maximum(hn, 0.0) + h


def _layer(h, Sp, degp, Ws, Wd, U1, U2, mb2d, ub2d):
    return pl.pallas_call(
        _layer_body,
        grid=(GRID,),
        in_specs=[
            pl.BlockSpec((BLK, D), lambda i: (i, 0)),
            pl.BlockSpec((BLK, D), lambda i: (i, 0)),
            pl.BlockSpec((BLK, D), lambda i: (i + GRID, 0)),
            pl.BlockSpec((BLK, D), lambda i: (i, 0)),
            pl.BlockSpec((BLK, D), lambda i: (i + GRID, 0)),
            pl.BlockSpec((D, D), lambda i: (0, 0)),
            pl.BlockSpec((D, D), lambda i: (0, 0)),
            pl.BlockSpec((D, D), lambda i: (0, 0)),
            pl.BlockSpec((D, D), lambda i: (0, 0)),
            pl.BlockSpec((1, D), lambda i: (0, 0)),
            pl.BlockSpec((1, D), lambda i: (0, 0)),
        ],
        out_specs=pl.BlockSpec((BLK, D), lambda i: (i, 0)),
        out_shape=jax.ShapeDtypeStruct((N, D), jnp.float32),
    )(h, Sp, Sp, degp, degp, Ws, Wd, U1, U2, mb2d, ub2d)


def _readout_body(h_ref, w1_ref, b1_ref, w2_ref, b2_ref, w3_ref, b3_ref,
                  o_ref):
    g = jnp.sum(h_ref[...], axis=0, keepdims=True) * (1.0 / N)
    o1 = jnp.maximum(jnp.dot(g, w1_ref[...],
                             preferred_element_type=jnp.float32,
                          precision=lax.Precision.HIGHEST)
                     + b1_ref[...], 0.0)
    o2 = jnp.maximum(jnp.dot(o1, w2_ref[...],
                             preferred_element_type=jnp.float32,
                          precision=lax.Precision.HIGHEST)
                     + b2_ref[...], 0.0)
    o3 = (jnp.dot(o2, w3_ref[...], preferred_element_type=jnp.float32,
                          precision=lax.Precision.HIGHEST)
          + b3_ref[...])
    o_ref[...] = jnp.broadcast_to(o3, (8, D))


def _readout(h, W1p, b1p, W2p, b2p, W3p, b3p):
    full = lambda shape: pl.BlockSpec(shape, lambda: tuple(0 for _ in shape))
    return pl.pallas_call(
        _readout_body,
        in_specs=[full((N, D)), full((D, D)), full((1, D)), full((D, D)),
                  full((1, D)), full((D, D)), full((1, D))],
        out_specs=full((8, D)),
        out_shape=jax.ShapeDtypeStruct((8, D), jnp.float32),
    )(h, W1p, b1p, W2p, b2p, W3p, b3p)


def kernel(atom_features, edge_indices, W_emb, b_emb, msg_W, msg_b,
           upd_W, upd_b, Wo1, bo1, Wo2, bo2, Wo3, bo3):
    src = edge_indices[0]
    dst = edge_indices[1]
    zrows = jnp.zeros((RPT, D), jnp.float32)

    degp = _deg_partials(dst, zrows, jnp.ones((CH, D), jnp.float32))
    h = _embed(atom_features, W_emb, b_emb.reshape(1, D))

    for i in range(msg_W.shape[0]):
        Sp = _segment_partials(h, src, dst, zrows)
        h = _layer(h, Sp, degp,
                   msg_W[i][:D], msg_W[i][D:],
                   upd_W[i][:D], upd_W[i][D:],
                   msg_b[i].reshape(1, D), upd_b[i].reshape(1, D))

    W1p = jnp.pad(Wo1, ((0, 0), (0, D - Wo1.shape[1])))
    b1p = jnp.pad(bo1, (0, D - bo1.shape[0])).reshape(1, D)
    W2p = jnp.pad(Wo2, ((0, D - Wo2.shape[0]), (0, D - Wo2.shape[1])))
    b2p = jnp.pad(bo2, (0, D - bo2.shape[0])).reshape(1, D)
    W3p = jnp.pad(Wo3, ((0, D - Wo3.shape[0]), (0, D - Wo3.shape[1])))
    b3p = jnp.pad(bo3, (0, D - bo3.shape[0])).reshape(1, D)
    o = _readout(h, W1p, b1p, W2p, b2p, W3p, b3p)
    return o[0:1, 0:1]
